# baseline (speedup 1.0000x reference)
import functools

import jax
import jax.numpy as jnp
from jax import lax
from jax.experimental import pallas as pl
from jax.experimental.pallas import tpu as pltpu
from jax.experimental.pallas import tpu_sc as plsc

KNN = 3
NQ = 8192
NC = 2048
DC = 256
DS = 128
DH = 256

TQ = 512
TM = 512


CW = 128


def _knn_body(ps_ref, bs_ref, pt_ref, bt_ref, xs_ref, w1b_ref, b1_ref,
              idx_ref, wn_ref, hp_ref):
    hp_ref[...] = (jnp.dot(xs_ref[...], w1b_ref[...],
                           preferred_element_type=jnp.float32) + b1_ref[...])
    bs = bs_ref[...]
    bt = bt_ref[...]
    b_min = bs_ref[0, 0]
    b_max = bs_ref[TQ - 1, 0]
    cnt_lt = jnp.sum((bt < b_min).astype(jnp.int32))
    cnt_le = jnp.sum((bt <= b_max).astype(jnp.int32))
    c_lo = cnt_lt // CW
    c_hi = (cnt_le + CW - 1) // CW
    ps = ps_ref[...]

    big = jnp.float32(1e30)
    init = (
        jnp.full((TQ, 1), big), jnp.zeros((TQ, 1), jnp.int32),
        jnp.full((TQ, 1), big), jnp.full((TQ, 1), 1, jnp.int32),
        jnp.full((TQ, 1), big), jnp.full((TQ, 1), 2, jnp.int32),
        jnp.zeros((TQ, 1), jnp.int32),
        jnp.zeros((TQ, 1), jnp.int32),
    )

    def chunk_body(c, carry):
        v0, i0, v1, i1, v2, i2, s_acc, e_acc = carry
        base = pl.multiple_of(c * CW, CW)
        ptc = pt_ref[:, pl.ds(base, CW)]
        btc = bt_ref[:, pl.ds(base, CW)]
        d2 = jnp.zeros((TQ, CW), jnp.float32)
        for cdim in range(3):
            diff = ps[:, cdim:cdim + 1] - ptc[cdim:cdim + 1, :]
            d2 = d2 + diff * diff
        lt = btc < bs
        eq = btc == bs
        s_acc = s_acc + jnp.sum(lt.astype(jnp.int32), axis=1, keepdims=True)
        e_acc = e_acc + jnp.sum((lt | eq).astype(jnp.int32), axis=1,
                                keepdims=True)
        d2 = jnp.where(eq, d2, big)
        cols = base + lax.broadcasted_iota(jnp.int32, (TQ, CW), 1)
        for _ in range(KNN):
            v = jnp.min(d2, axis=1, keepdims=True)
            i = jnp.min(jnp.where(d2 == v, cols, NC), axis=1, keepdims=True)
            d2 = jnp.where(cols == i, jnp.float32(1e38), d2)
            lt0 = v < v0
            lt1 = v < v1
            lt2 = v < v2
            v2, i2 = (jnp.where(lt2, jnp.where(lt1, v1, v), v2),
                      jnp.where(lt2, jnp.where(lt1, i1, i), i2))
            v1, i1 = (jnp.where(lt1, jnp.where(lt0, v0, v), v1),
                      jnp.where(lt1, jnp.where(lt0, i0, i), i1))
            v0, i0 = jnp.where(lt0, v, v0), jnp.where(lt0, i, i0)
        return (v0, i0, v1, i1, v2, i2, s_acc, e_acc)

    v0, i0, v1, i1, v2, i2, s_acc, e_acc = lax.fori_loop(
        c_lo, c_hi, chunk_body, init)
    s = c_lo * CW + s_acc
    e = c_lo * CW + e_acc
    for j, (vj, ij) in enumerate(((v0, i0), (v1, i1), (v2, i2))):
        fj = jnp.where(j < s, j, e + (j - s))
        ij = jnp.where(vj >= big, fj, ij)
        if j == 0:
            i0 = ij
        elif j == 1:
            i1 = ij
        else:
            i2 = ij
    w = [1.0 / jnp.maximum(v, jnp.float32(1e-16)) for v in (v0, v1, v2)]
    den = w[0] + w[1] + w[2]
    wn = [wi / den for wi in w]
    idx_ref[...] = jnp.concatenate([i0, i1, i2], axis=1)
    wn_ref[...] = jnp.concatenate(
        wn + [jnp.zeros((TQ, 8 - KNN), jnp.float32)], axis=1)


def _knn_call(pos_skip, bs_col, pos_t, batch_row, x_skip, W1b, b1r):
    return pl.pallas_call(
        _knn_body,
        grid=(NQ // TQ,),
        in_specs=[
            pl.BlockSpec((TQ, 3), lambda i: (i, 0)),
            pl.BlockSpec((TQ, 1), lambda i: (i, 0)),
            pl.BlockSpec((3, NC), lambda i: (0, 0)),
            pl.BlockSpec((1, NC), lambda i: (0, 0)),
            pl.BlockSpec((TQ, DS), lambda i: (i, 0)),
            pl.BlockSpec((DS, DH), lambda i: (0, 0)),
            pl.BlockSpec((1, DH), lambda i: (0, 0)),
        ],
        out_specs=[
            pl.BlockSpec((TQ, KNN), lambda i: (i, 0)),
            pl.BlockSpec((TQ, 8), lambda i: (i, 0)),
            pl.BlockSpec((TQ, DH), lambda i: (i, 0)),
        ],
        out_shape=[
            jax.ShapeDtypeStruct((NQ, KNN), jnp.int32),
            jax.ShapeDtypeStruct((NQ, 8), jnp.float32),
            jax.ShapeDtypeStruct((NQ, DH), jnp.float32),
        ],
    )(pos_skip, bs_col, pos_t, batch_row, x_skip, W1b, b1r)


NROWS = KNN * NQ
NW = 32
RPW = NROWS // NW
CH = 128
NCHUNK = RPW // CH


def _gather_body(x_hbm, idx_hbm, out_hbm, idx0, idx1, rows0, rows1,
                 sem0, sem1):
    wid = lax.axis_index("s") * 2 + lax.axis_index("c")
    base = pl.multiple_of(wid * RPW, CH)
    idx_v = (idx0, idx1)
    rows_v = (rows0, rows1)
    sems = (sem0, sem1)
    handles = [None, None]
    pltpu.sync_copy(idx_hbm.at[pl.ds(base, CH)], idx0)
    handles[0] = pltpu.async_copy(x_hbm.at[idx0], rows0, sem0)
    for c in range(NCHUNK):
        cur = c % 2
        nxt = (c + 1) % 2
        if c + 1 < NCHUNK:
            off_n = pl.multiple_of(base + (c + 1) * CH, CH)
            pltpu.sync_copy(idx_hbm.at[pl.ds(off_n, CH)], idx_v[nxt])
            handles[nxt] = pltpu.async_copy(x_hbm.at[idx_v[nxt]],
                                            rows_v[nxt], sems[nxt])
        off = pl.multiple_of(base + c * CH, CH)
        handles[cur].wait()
        pltpu.sync_copy(rows_v[cur], out_hbm.at[pl.ds(off, CH)])


def _gather_call(x, idx_flat):
    mesh = plsc.VectorSubcoreMesh(core_axis_name="c", subcore_axis_name="s")
    fn = functools.partial(
        pl.kernel,
        mesh=mesh,
        out_type=jax.ShapeDtypeStruct((NROWS, DC), jnp.float32),
        scratch_types=[
            pltpu.VMEM((CH,), jnp.int32),
            pltpu.VMEM((CH,), jnp.int32),
            pltpu.VMEM((CH, DC), jnp.float32),
            pltpu.VMEM((CH, DC), jnp.float32),
            pltpu.SemaphoreType.DMA,
            pltpu.SemaphoreType.DMA,
        ],
    )(_gather_body)
    return fn(x, idx_flat)


def _mlp_body(f_ref, wn_ref, hp_ref, w1a_ref, w2_ref, b2_ref, out_ref):
    f = f_ref[...]
    y = (wn_ref[:, 0:1] * f[:, 0 * DC:1 * DC]
         + wn_ref[:, 1:2] * f[:, 1 * DC:2 * DC]
         + wn_ref[:, 2:3] * f[:, 2 * DC:3 * DC])
    h = jnp.maximum(
        jnp.dot(y, w1a_ref[...], preferred_element_type=jnp.float32)
        + hp_ref[...], 0.0)
    out_ref[...] = (jnp.dot(h, w2_ref[...], preferred_element_type=jnp.float32)
                    + b2_ref[...])


def _mlp_call(feats2, wn8, h_part, W1a, W2, b2r):
    return pl.pallas_call(
        _mlp_body,
        grid=(NQ // TM,),
        in_specs=[
            pl.BlockSpec((TM, KNN * DC), lambda i: (i, 0)),
            pl.BlockSpec((TM, 8), lambda i: (i, 0)),
            pl.BlockSpec((TM, DH), lambda i: (i, 0)),
            pl.BlockSpec((DC, DH), lambda i: (0, 0)),
            pl.BlockSpec((DH, DH), lambda i: (0, 0)),
            pl.BlockSpec((1, DH), lambda i: (0, 0)),
        ],
        out_specs=pl.BlockSpec((TM, DH), lambda i: (i, 0)),
        out_shape=jax.ShapeDtypeStruct((NQ, DH), jnp.float32),
    )(feats2, wn8, h_part, W1a, W2, b2r)


def kernel(x, pos, batch, x_skip, pos_skip, batch_skip, W1, b1, W2, b2):
    pos_t = pos.T
    batch_row = batch.reshape(1, NC)
    bs_col = batch_skip.reshape(NQ, 1)
    b1r = b1.reshape(1, DH)
    idx3, wn8, h_part = _knn_call(pos_skip, bs_col, pos_t, batch_row,
                                  x_skip, W1[DC:], b1r)
    idx_flat = idx3.reshape(-1)
    feats = _gather_call(x, idx_flat)
    feats2 = feats.reshape(NQ, KNN * DC)
    out = _mlp_call(feats2, wn8, h_part, W1[:DC], W2, b2.reshape(1, DH))
    return (out, pos_skip, batch_skip)


# speedup vs baseline: 1.1951x; 1.1951x over previous
import functools

import jax
import jax.numpy as jnp
from jax import lax
from jax.experimental import pallas as pl
from jax.experimental.pallas import tpu as pltpu
from jax.experimental.pallas import tpu_sc as plsc

KNN = 3
NQ = 8192
NC = 2048
DC = 256
DS = 128
DH = 256

TQ = 512
TM = 512


def _knn_body(qa_ref, bs_ref, pb_ref, bt_ref, xs_ref, w1b_ref, b1_ref,
              idx_ref, wn_ref, hp_ref):
    hp_ref[...] = (jnp.dot(xs_ref[...], w1b_ref[...],
                           preferred_element_type=jnp.float32) + b1_ref[...])
    d2 = jnp.dot(qa_ref[...], pb_ref[...],
                 preferred_element_type=jnp.float32)
    mask = bs_ref[:, :] != bt_ref[:, :]
    d2 = jnp.where(mask, jnp.float32(1e30), d2)
    cols = lax.broadcasted_iota(jnp.int32, (TQ, NC), 1)
    vals, idxs = [], []
    for k in range(KNN):
        v = jnp.min(d2, axis=1, keepdims=True)
        i = jnp.min(jnp.where(d2 == v, cols, NC), axis=1, keepdims=True)
        vals.append(v)
        idxs.append(i)
        if k + 1 < KNN:
            d2 = jnp.where(cols == i, jnp.float32(1e38), d2)
    w = [1.0 / jnp.maximum(v, jnp.float32(1e-16)) for v in vals]
    den = w[0] + w[1] + w[2]
    wn = [wi / den for wi in w]
    idx_ref[...] = jnp.concatenate(idxs, axis=1)
    wn_ref[...] = jnp.concatenate(
        wn + [jnp.zeros((TQ, 8 - KNN), jnp.float32)], axis=1)


def _knn_call(q_aug, bs_col, p_aug, batch_row, x_skip, W1b, b1r):
    return pl.pallas_call(
        _knn_body,
        grid=(NQ // TQ,),
        in_specs=[
            pl.BlockSpec((TQ, 8), lambda i: (i, 0)),
            pl.BlockSpec((TQ, 1), lambda i: (i, 0)),
            pl.BlockSpec((8, NC), lambda i: (0, 0)),
            pl.BlockSpec((1, NC), lambda i: (0, 0)),
            pl.BlockSpec((TQ, DS), lambda i: (i, 0)),
            pl.BlockSpec((DS, DH), lambda i: (0, 0)),
            pl.BlockSpec((1, DH), lambda i: (0, 0)),
        ],
        out_specs=[
            pl.BlockSpec((TQ, KNN), lambda i: (i, 0)),
            pl.BlockSpec((TQ, 8), lambda i: (i, 0)),
            pl.BlockSpec((TQ, DH), lambda i: (i, 0)),
        ],
        out_shape=[
            jax.ShapeDtypeStruct((NQ, KNN), jnp.int32),
            jax.ShapeDtypeStruct((NQ, 8), jnp.float32),
            jax.ShapeDtypeStruct((NQ, DH), jnp.float32),
        ],
    )(q_aug, bs_col, p_aug, batch_row, x_skip, W1b, b1r)


NROWS = KNN * NQ
NW = 32
RPW = NROWS // NW
CH = 128
NCHUNK = RPW // CH


def _gather_body(x_hbm, idx_hbm, out_hbm, idx0, idx1, rows0, rows1,
                 sem0, sem1):
    wid = lax.axis_index("s") * 2 + lax.axis_index("c")
    base = pl.multiple_of(wid * RPW, CH)
    idx_v = (idx0, idx1)
    rows_v = (rows0, rows1)
    sems = (sem0, sem1)
    handles = [None, None]
    pltpu.sync_copy(idx_hbm.at[pl.ds(base, CH)], idx0)
    handles[0] = pltpu.async_copy(x_hbm.at[idx0], rows0, sem0)
    for c in range(NCHUNK):
        cur = c % 2
        nxt = (c + 1) % 2
        if c + 1 < NCHUNK:
            off_n = pl.multiple_of(base + (c + 1) * CH, CH)
            pltpu.sync_copy(idx_hbm.at[pl.ds(off_n, CH)], idx_v[nxt])
            handles[nxt] = pltpu.async_copy(x_hbm.at[idx_v[nxt]],
                                            rows_v[nxt], sems[nxt])
        off = pl.multiple_of(base + c * CH, CH)
        handles[cur].wait()
        pltpu.sync_copy(rows_v[cur], out_hbm.at[pl.ds(off, CH)])


def _gather_call(x, idx_flat):
    mesh = plsc.VectorSubcoreMesh(core_axis_name="c", subcore_axis_name="s")
    fn = functools.partial(
        pl.kernel,
        mesh=mesh,
        out_type=jax.ShapeDtypeStruct((NROWS, DC), jnp.float32),
        scratch_types=[
            pltpu.VMEM((CH,), jnp.int32),
            pltpu.VMEM((CH,), jnp.int32),
            pltpu.VMEM((CH, DC), jnp.float32),
            pltpu.VMEM((CH, DC), jnp.float32),
            pltpu.SemaphoreType.DMA,
            pltpu.SemaphoreType.DMA,
        ],
    )(_gather_body)
    return fn(x, idx_flat)


def _mlp_body(f_ref, wn_ref, hp_ref, w1a_ref, w2_ref, b2_ref, out_ref):
    f = f_ref[...]
    y = (wn_ref[:, 0:1] * f[:, 0 * DC:1 * DC]
         + wn_ref[:, 1:2] * f[:, 1 * DC:2 * DC]
         + wn_ref[:, 2:3] * f[:, 2 * DC:3 * DC])
    h = jnp.maximum(
        jnp.dot(y, w1a_ref[...], preferred_element_type=jnp.float32)
        + hp_ref[...], 0.0)
    out_ref[...] = (jnp.dot(h, w2_ref[...], preferred_element_type=jnp.float32)
                    + b2_ref[...])


def _mlp_call(feats2, wn8, h_part, W1a, W2, b2r):
    return pl.pallas_call(
        _mlp_body,
        grid=(NQ // TM,),
        in_specs=[
            pl.BlockSpec((TM, KNN * DC), lambda i: (i, 0)),
            pl.BlockSpec((TM, 8), lambda i: (i, 0)),
            pl.BlockSpec((TM, DH), lambda i: (i, 0)),
            pl.BlockSpec((DC, DH), lambda i: (0, 0)),
            pl.BlockSpec((DH, DH), lambda i: (0, 0)),
            pl.BlockSpec((1, DH), lambda i: (0, 0)),
        ],
        out_specs=pl.BlockSpec((TM, DH), lambda i: (i, 0)),
        out_shape=jax.ShapeDtypeStruct((NQ, DH), jnp.float32),
    )(feats2, wn8, h_part, W1a, W2, b2r)


def kernel(x, pos, batch, x_skip, pos_skip, batch_skip, W1, b1, W2, b2):
    q_aug = jnp.concatenate(
        [pos_skip,
         jnp.sum(pos_skip * pos_skip, axis=1, keepdims=True),
         jnp.ones((NQ, 1), jnp.float32),
         jnp.zeros((NQ, 3), jnp.float32)], axis=1)
    p_aug = jnp.concatenate(
        [-2.0 * pos.T,
         jnp.ones((1, NC), jnp.float32),
         jnp.sum(pos * pos, axis=1).reshape(1, NC),
         jnp.zeros((3, NC), jnp.float32)], axis=0)
    batch_row = batch.reshape(1, NC)
    bs_col = batch_skip.reshape(NQ, 1)
    b1r = b1.reshape(1, DH)
    idx3, wn8, h_part = _knn_call(q_aug, bs_col, p_aug, batch_row,
                                  x_skip, W1[DC:], b1r)
    idx_flat = idx3.reshape(-1)
    feats = _gather_call(x, idx_flat)
    feats2 = feats.reshape(NQ, KNN * DC)
    out = _mlp_call(feats2, wn8, h_part, W1[:DC], W2, b2.reshape(1, DH))
    return (out, pos_skip, batch_skip)
